# baseline (speedup 1.0000x reference)
import functools

import jax
import jax.numpy as jnp
from jax import lax
from jax.experimental import pallas as pl
from jax.experimental.pallas import tpu as pltpu

VOCAB = 68
D = 1024
L = 2048
B = 32
VP = 128
TB = 2048
NBLK = B * L // TB


def _positional_encoding():
    pos = jnp.arange(L, dtype=jnp.float32)[:, None]
    i = jnp.arange(0, D, 2, dtype=jnp.float32)
    denom = jnp.power(10000.0, i / D)
    ang = pos / denom[None, :]
    return jnp.stack([jnp.sin(ang), jnp.cos(ang)], axis=2).reshape(L, D)


def _tc_body(x_ref, hi_ref, lo_ref, pe_ref, o_ref):
    xv = x_ref[...]
    iot = lax.broadcasted_iota(jnp.int32, (TB, VP), 1)
    onehot = (iot == xv).astype(jnp.bfloat16)
    emb = jnp.dot(onehot, hi_ref[...], preferred_element_type=jnp.float32)
    emb += jnp.dot(onehot, lo_ref[...], preferred_element_type=jnp.float32)
    o_ref[...] = emb + pe_ref[...]


def kernel(x, table):
    pe = _positional_encoding()
    x_col = x.reshape(B * L, 1).astype(jnp.int32)
    tab_pad = jnp.zeros((VP, D), jnp.float32).at[:VOCAB].set(table)
    hi = tab_pad.astype(jnp.bfloat16)
    lo = (tab_pad - hi.astype(jnp.float32)).astype(jnp.bfloat16)

    out = pl.pallas_call(
        _tc_body,
        grid=(NBLK,),
        in_specs=[
            pl.BlockSpec((TB, 1), lambda j: (j, 0)),
            pl.BlockSpec((VP, D), lambda j: (0, 0)),
            pl.BlockSpec((VP, D), lambda j: (0, 0)),
            pl.BlockSpec((TB, D), lambda j: (0, 0)),
        ],
        out_specs=pl.BlockSpec((TB, D), lambda j: (j, 0)),
        out_shape=jax.ShapeDtypeStruct((B * L, D), jnp.float32),
    )(x_col, hi, lo, pe)
    return out.reshape(B, L, D)
